# Initial kernel scaffold; baseline (speedup 1.0000x reference)
import functools

import jax
import jax.numpy as jnp
from jax import lax
from jax.experimental import pallas as pl
from jax.experimental.pallas import tpu as pltpu
from jax.experimental.pallas import tpu_sc as plsc

F32 = jnp.float32

NC = 2
NS = 16
NW = NC * NS
WIN = 128
BLK = 512
CH = 32


def _cdiv(a, b):
    return (a + b - 1) // b


@functools.lru_cache(maxsize=None)
def _sc_hist(n_pad, r):
    mesh = plsc.VectorSubcoreMesh(core_axis_name="c", subcore_axis_name="s")
    rpw = n_pad // NS
    zrows = rpw // 4

    def body(dst_hbm, out_hbm, dst_v, ones_v, zbuf, acc, sem):
        c = lax.axis_index("c")
        s = lax.axis_index("s")
        wid = c * NS + s
        pltpu.sync_copy(dst_hbm.at[wid], dst_v)

        @pl.loop(0, WIN)
        def _(i):
            ones_v[i, pl.ds(0, 16)] = jnp.ones((16,), F32)

        @pl.loop(0, zrows)
        def _(i):
            zbuf[i, pl.ds(0, 16)] = jnp.zeros((16,), F32)

        for q in range(4):
            pltpu.sync_copy(zbuf, acc.at[pl.ds(s * rpw + q * zrows, zrows)])
        plsc.subcore_barrier()

        @pl.loop(0, r)
        def _(j):
            pltpu.async_copy(ones_v, acc.at[dst_v.at[j]], sem, add=True).wait()

        plsc.subcore_barrier()
        pltpu.sync_copy(acc.at[pl.ds(s * rpw, rpw)],
                        out_hbm.at[c, pl.ds(s * rpw, rpw)])

    return pl.kernel(
        body,
        mesh=mesh,
        out_type=jax.ShapeDtypeStruct((NC, n_pad, 16), F32),
        scratch_types=[
            pltpu.VMEM((r, WIN), jnp.int32),
            pltpu.VMEM((WIN, 16), F32),
            pltpu.VMEM((zrows, 16), F32),
            pltpu.VMEM_SHARED((n_pad, 16), F32),
            pltpu.SemaphoreType.DMA,
        ],
    )


@functools.lru_cache(maxsize=None)
def _sc_scatter(n_chunks, n_pad, r):
    mesh = plsc.VectorSubcoreMesh(core_axis_name="c", subcore_axis_name="s")
    rpw = n_pad // NS
    zrows = rpw // 4

    def body(src_hbm, dst_hbm, *rest):
        g_refs = rest[:n_chunks]
        out_refs = rest[n_chunks:2 * n_chunks]
        src_v, dst_v, rows_v, zbuf, acc, sem_g, sem_s = rest[2 * n_chunks:]
        c = lax.axis_index("c")
        s = lax.axis_index("s")
        wid = c * NS + s
        pltpu.sync_copy(src_hbm.at[wid], src_v)
        pltpu.sync_copy(dst_hbm.at[wid], dst_v)

        @pl.loop(0, zrows)
        def _(i):
            zbuf[i, pl.ds(0, 16)] = jnp.zeros((16,), F32)
            zbuf[i, pl.ds(16, 16)] = jnp.zeros((16,), F32)

        for ci in range(n_chunks):
            for q in range(4):
                pltpu.sync_copy(zbuf, acc.at[pl.ds(s * rpw + q * zrows, zrows)])
            plsc.subcore_barrier()

            @pl.loop(0, r)
            def _(j):
                pltpu.async_copy(g_refs[ci].at[src_v.at[j]], rows_v, sem_g).wait()
                pltpu.async_copy(rows_v, acc.at[dst_v.at[j]], sem_s, add=True).wait()

            plsc.subcore_barrier()
            pltpu.sync_copy(acc.at[pl.ds(s * rpw, rpw)],
                            out_refs[ci].at[c, pl.ds(s * rpw, rpw)])
            if ci + 1 < n_chunks:
                plsc.subcore_barrier()

    return pl.kernel(
        body,
        mesh=mesh,
        out_type=tuple(jax.ShapeDtypeStruct((NC, n_pad, CH), F32)
                       for _ in range(n_chunks)),
        scratch_types=[
            pltpu.VMEM((r, WIN), jnp.int32),
            pltpu.VMEM((r, WIN), jnp.int32),
            pltpu.VMEM((WIN, CH), F32),
            pltpu.VMEM((zrows, CH), F32),
            pltpu.VMEM_SHARED((n_pad, CH), F32),
            pltpu.SemaphoreType.DMA,
            pltpu.SemaphoreType.DMA,
        ],
    )


def _dot(a, b):
    return lax.dot_general(a, b, (((1,), (0,)), ((), ())),
                           precision=lax.Precision.HIGHEST,
                           preferred_element_type=F32)


def _tc_prep(hist, x_p, n_real):
    n_pad, cin = x_p.shape
    nb = n_pad // BLK
    nch = cin // CH

    def body(hist_ref, x_ref, dis_ref, *g_refs):
        i = pl.program_id(0)
        deg = hist_ref[0] + hist_ref[1] + 1.0
        row = i * BLK + lax.broadcasted_iota(jnp.int32, (BLK, 16), 0)
        dis = jnp.where(row < n_real, lax.rsqrt(deg), 0.0)
        dis_col = dis[:, 0:1]
        dis_ref[...] = dis_col
        xs = x_ref[...] * dis_col
        for ci in range(nch):
            g_refs[ci][...] = xs[:, ci * CH:(ci + 1) * CH]

    return pl.pallas_call(
        body,
        grid=(nb,),
        in_specs=[
            pl.BlockSpec((NC, BLK, 16), lambda i: (0, i, 0)),
            pl.BlockSpec((BLK, cin), lambda i: (i, 0)),
        ],
        out_specs=[pl.BlockSpec((BLK, 1), lambda i: (i, 0))] +
                  [pl.BlockSpec((BLK, CH), lambda i: (i, 0))] * nch,
        out_shape=[jax.ShapeDtypeStruct((n_pad, 1), F32)] +
                  [jax.ShapeDtypeStruct((n_pad, CH), F32)] * nch,
    )(hist, x_p)


def _combine(s_refs):
    return jnp.concatenate([ref[0] + ref[1] for ref in s_refs], axis=1)


def _tc_layer1(s0, x_p, dis, W1, b1, W2):
    n_pad, cin = x_p.shape
    hid = W1.shape[1]
    mid = W2.shape[1]
    nb = n_pad // BLK
    nch_in = len(s0)
    nch_out = mid // CH

    def body(*refs):
        s_refs = refs[:nch_in]
        x_ref, dis_ref, w1_ref, b1_ref, w2_ref = refs[nch_in:nch_in + 5]
        m1_ref = refs[nch_in + 5]
        g_refs = refs[nch_in + 6:]
        dis = dis_ref[...]
        p0 = dis * _combine(s_refs) + (dis * dis) * x_ref[...]
        h1 = jnp.maximum(_dot(p0, w1_ref[...]) + b1_ref[...], 0.0)
        m1 = _dot(h1, w2_ref[...])
        m1_ref[...] = m1
        g1 = dis * m1
        for ci in range(nch_out):
            g_refs[ci][...] = g1[:, ci * CH:(ci + 1) * CH]

    return pl.pallas_call(
        body,
        grid=(nb,),
        in_specs=[pl.BlockSpec((NC, BLK, CH), lambda i: (0, i, 0))] * nch_in + [
            pl.BlockSpec((BLK, cin), lambda i: (i, 0)),
            pl.BlockSpec((BLK, 1), lambda i: (i, 0)),
            pl.BlockSpec((cin, hid), lambda i: (0, 0)),
            pl.BlockSpec((1, hid), lambda i: (0, 0)),
            pl.BlockSpec((hid, mid), lambda i: (0, 0)),
        ],
        out_specs=[pl.BlockSpec((BLK, mid), lambda i: (i, 0))] +
                  [pl.BlockSpec((BLK, CH), lambda i: (i, 0))] * nch_out,
        out_shape=[jax.ShapeDtypeStruct((n_pad, mid), F32)] +
                  [jax.ShapeDtypeStruct((n_pad, CH), F32)] * nch_out,
    )(*s0, x_p, dis, W1, b1.reshape(1, -1), W2)


def _tc_layer2(s1, m1, dis, b2, W3):
    n_pad, mid = m1.shape
    out_dim = W3.shape[1]
    nb = n_pad // BLK
    nch_in = len(s1)
    nch_out = out_dim // CH

    def body(*refs):
        s_refs = refs[:nch_in]
        m1_ref, dis_ref, b2_ref, w3_ref = refs[nch_in:nch_in + 4]
        m2_ref = refs[nch_in + 4]
        g_refs = refs[nch_in + 5:]
        dis = dis_ref[...]
        h2 = jnp.maximum(dis * _combine(s_refs) +
                         (dis * dis) * m1_ref[...] + b2_ref[...], 0.0)
        m2 = _dot(h2, w3_ref[...])
        m2_ref[...] = m2
        g2 = dis * m2
        for ci in range(nch_out):
            g_refs[ci][...] = g2[:, ci * CH:(ci + 1) * CH]

    return pl.pallas_call(
        body,
        grid=(nb,),
        in_specs=[pl.BlockSpec((NC, BLK, CH), lambda i: (0, i, 0))] * nch_in + [
            pl.BlockSpec((BLK, mid), lambda i: (i, 0)),
            pl.BlockSpec((BLK, 1), lambda i: (i, 0)),
            pl.BlockSpec((1, mid), lambda i: (0, 0)),
            pl.BlockSpec((mid, out_dim), lambda i: (0, 0)),
        ],
        out_specs=[pl.BlockSpec((BLK, out_dim), lambda i: (i, 0))] +
                  [pl.BlockSpec((BLK, CH), lambda i: (i, 0))] * nch_out,
        out_shape=[jax.ShapeDtypeStruct((n_pad, out_dim), F32)] +
                  [jax.ShapeDtypeStruct((n_pad, CH), F32)] * nch_out,
    )(*s1, m1, dis, b2.reshape(1, -1), W3)


def _tc_final(s2, m2, dis, b3):
    n_pad, out_dim = m2.shape
    nb = n_pad // BLK
    nch_in = len(s2)

    def body(*refs):
        s_refs = refs[:nch_in]
        m2_ref, dis_ref, b3_ref, out_ref = refs[nch_in:]
        dis = dis_ref[...]
        out_ref[...] = jnp.maximum(dis * _combine(s_refs) +
                                   (dis * dis) * m2_ref[...] + b3_ref[...], 0.0)

    return pl.pallas_call(
        body,
        grid=(nb,),
        in_specs=[pl.BlockSpec((NC, BLK, CH), lambda i: (0, i, 0))] * nch_in + [
            pl.BlockSpec((BLK, out_dim), lambda i: (i, 0)),
            pl.BlockSpec((BLK, 1), lambda i: (i, 0)),
            pl.BlockSpec((1, out_dim), lambda i: (0, 0)),
        ],
        out_specs=pl.BlockSpec((BLK, out_dim), lambda i: (i, 0)),
        out_shape=jax.ShapeDtypeStruct((n_pad, out_dim), F32),
    )(*s2, m2, dis, b3.reshape(1, -1))


def kernel(x, edge_index, W1, b1, W2, b2, W3, b3):
    n, cin = x.shape
    e = edge_index.shape[1]

    r = _cdiv(e, NW * WIN)
    e_pad = NW * r * WIN
    n_pad = (_cdiv(n + 1, BLK)) * BLK

    src = edge_index[0].astype(jnp.int32)
    dst = edge_index[1].astype(jnp.int32)
    pad = jnp.full((e_pad - e,), n, jnp.int32)
    src3 = jnp.concatenate([src, pad]).reshape(NW, r, WIN)
    dst3 = jnp.concatenate([dst, pad]).reshape(NW, r, WIN)
    x_p = jnp.pad(x, ((0, n_pad - n), (0, 0)))

    hist = _sc_hist(n_pad, r)(dst3)
    dis, *g0 = _tc_prep(hist, x_p, n)
    s0 = _sc_scatter(len(g0), n_pad, r)(src3, dst3, *g0)
    m1, *g1 = _tc_layer1(s0, x_p, dis, W1, b1, W2)
    s1 = _sc_scatter(len(g1), n_pad, r)(src3, dst3, *g1)
    m2, *g2 = _tc_layer2(s1, m1, dis, b2, W3)
    s2 = _sc_scatter(len(g2), n_pad, r)(src3, dst3, *g2)
    out = _tc_final(s2, m2, dis, b3)
    return out[:n]


# speedup vs baseline: 5.9619x; 5.9619x over previous
import functools

import jax
import jax.numpy as jnp
from jax import lax
from jax.experimental import pallas as pl
from jax.experimental.pallas import tpu as pltpu
from jax.experimental.pallas import tpu_sc as plsc

F32 = jnp.float32

NC = 2
NS = 16
NW = NC * NS
WIN = 128
BLK = 512
CH = 16


def _cdiv(a, b):
    return (a + b - 1) // b


@functools.lru_cache(maxsize=None)
def _sc_hist(n_pad, r):
    mesh = plsc.VectorSubcoreMesh(core_axis_name="c", subcore_axis_name="s")
    rpw = n_pad // NS
    zrows = rpw // 4

    def body(dst_hbm, out_hbm, dst_v, ones_v, zbuf, acc, sem):
        c = lax.axis_index("c")
        s = lax.axis_index("s")
        wid = c * NS + s
        pltpu.sync_copy(dst_hbm.at[wid], dst_v)

        @pl.loop(0, WIN)
        def _(i):
            ones_v[i, pl.ds(0, 16)] = jnp.ones((16,), F32)

        @pl.loop(0, zrows)
        def _(i):
            zbuf[i, pl.ds(0, 16)] = jnp.zeros((16,), F32)

        for q in range(4):
            pltpu.sync_copy(zbuf, acc.at[pl.ds(s * rpw + q * zrows, zrows)])
        plsc.subcore_barrier()

        @pl.loop(0, r)
        def _(j):
            pltpu.async_copy(ones_v, acc.at[dst_v.at[j]], sem, add=True).wait()

        plsc.subcore_barrier()
        pltpu.sync_copy(acc.at[pl.ds(s * rpw, rpw)],
                        out_hbm.at[c, pl.ds(s * rpw, rpw)])

    return pl.kernel(
        body,
        mesh=mesh,
        compiler_params=pltpu.CompilerParams(use_tc_tiling_on_sc=False),
        out_type=jax.ShapeDtypeStruct((NC, n_pad, 16), F32),
        scratch_types=[
            pltpu.VMEM((r, WIN), jnp.int32),
            pltpu.VMEM((WIN, 16), F32),
            pltpu.VMEM((zrows, 16), F32),
            pltpu.VMEM_SHARED((n_pad, 16), F32),
            pltpu.SemaphoreType.DMA,
        ],
    )


@functools.lru_cache(maxsize=None)
def _sc_scatter(n_chunks, n_pad, r):
    mesh = plsc.VectorSubcoreMesh(core_axis_name="c", subcore_axis_name="s")
    rpw = n_pad // NS
    zrows = rpw // 4

    def body(src_hbm, dst_hbm, *rest):
        g_refs = rest[:n_chunks]
        out_refs = rest[n_chunks:2 * n_chunks]
        src_v, dst_v, rows_v, zbuf, acc, sem_g, sem_s = rest[2 * n_chunks:]
        c = lax.axis_index("c")
        s = lax.axis_index("s")
        wid = c * NS + s
        pltpu.sync_copy(src_hbm.at[wid], src_v)
        pltpu.sync_copy(dst_hbm.at[wid], dst_v)

        @pl.loop(0, zrows)
        def _(i):
            for h in range(CH // 16):
                zbuf[i, pl.ds(h * 16, 16)] = jnp.zeros((16,), F32)

        for ci in range(n_chunks):
            for q in range(4):
                pltpu.sync_copy(zbuf, acc.at[pl.ds(s * rpw + q * zrows, zrows)])
            plsc.subcore_barrier()

            @pl.loop(0, r)
            def _(j):
                pltpu.async_copy(g_refs[ci].at[src_v.at[j]], rows_v, sem_g).wait()
                pltpu.async_copy(rows_v, acc.at[dst_v.at[j]], sem_s, add=True).wait()

            plsc.subcore_barrier()
            pltpu.sync_copy(acc.at[pl.ds(s * rpw, rpw)],
                            out_refs[ci].at[c, pl.ds(s * rpw, rpw)])
            if ci + 1 < n_chunks:
                plsc.subcore_barrier()

    return pl.kernel(
        body,
        mesh=mesh,
        compiler_params=pltpu.CompilerParams(use_tc_tiling_on_sc=False),
        out_type=tuple(jax.ShapeDtypeStruct((NC, n_pad, CH), F32)
                       for _ in range(n_chunks)),
        scratch_types=[
            pltpu.VMEM((r, WIN), jnp.int32),
            pltpu.VMEM((r, WIN), jnp.int32),
            pltpu.VMEM((WIN, CH), F32),
            pltpu.VMEM((zrows, CH), F32),
            pltpu.VMEM_SHARED((n_pad, CH), F32),
            pltpu.SemaphoreType.DMA,
            pltpu.SemaphoreType.DMA,
        ],
    )


def _dot(a, b):
    return lax.dot_general(a, b, (((1,), (0,)), ((), ())),
                           precision=lax.Precision.HIGHEST,
                           preferred_element_type=F32)


def _tc_prep(hist, x_p, n_real):
    n_pad, cin = x_p.shape
    nb = n_pad // BLK
    nch = cin // CH

    def body(hist_ref, x_ref, dis_ref, *g_refs):
        i = pl.program_id(0)
        deg = hist_ref[0] + hist_ref[1] + 1.0
        row = i * BLK + lax.broadcasted_iota(jnp.int32, (BLK, 16), 0)
        dis = jnp.where(row < n_real, lax.rsqrt(deg), 0.0)
        dis_col = dis[:, 0:1]
        dis_ref[...] = dis_col
        xs = x_ref[...] * dis_col
        for ci in range(nch):
            g_refs[ci][...] = xs[:, ci * CH:(ci + 1) * CH]

    return pl.pallas_call(
        body,
        grid=(nb,),
        in_specs=[
            pl.BlockSpec((NC, BLK, 16), lambda i: (0, i, 0)),
            pl.BlockSpec((BLK, cin), lambda i: (i, 0)),
        ],
        out_specs=[pl.BlockSpec((BLK, 1), lambda i: (i, 0))] +
                  [pl.BlockSpec((BLK, CH), lambda i: (i, 0))] * nch,
        out_shape=[jax.ShapeDtypeStruct((n_pad, 1), F32)] +
                  [jax.ShapeDtypeStruct((n_pad, CH), F32)] * nch,
    )(hist, x_p)


def _combine(s_refs):
    return jnp.concatenate([ref[0] + ref[1] for ref in s_refs], axis=1)


def _tc_layer1(s0, x_p, dis, W1, b1, W2):
    n_pad, cin = x_p.shape
    hid = W1.shape[1]
    mid = W2.shape[1]
    nb = n_pad // BLK
    nch_in = len(s0)
    nch_out = mid // CH

    def body(*refs):
        s_refs = refs[:nch_in]
        x_ref, dis_ref, w1_ref, b1_ref, w2_ref = refs[nch_in:nch_in + 5]
        m1_ref = refs[nch_in + 5]
        g_refs = refs[nch_in + 6:]
        dis = dis_ref[...]
        p0 = dis * _combine(s_refs) + (dis * dis) * x_ref[...]
        h1 = jnp.maximum(_dot(p0, w1_ref[...]) + b1_ref[...], 0.0)
        m1 = _dot(h1, w2_ref[...])
        m1_ref[...] = m1
        g1 = dis * m1
        for ci in range(nch_out):
            g_refs[ci][...] = g1[:, ci * CH:(ci + 1) * CH]

    return pl.pallas_call(
        body,
        grid=(nb,),
        in_specs=[pl.BlockSpec((NC, BLK, CH), lambda i: (0, i, 0))] * nch_in + [
            pl.BlockSpec((BLK, cin), lambda i: (i, 0)),
            pl.BlockSpec((BLK, 1), lambda i: (i, 0)),
            pl.BlockSpec((cin, hid), lambda i: (0, 0)),
            pl.BlockSpec((1, hid), lambda i: (0, 0)),
            pl.BlockSpec((hid, mid), lambda i: (0, 0)),
        ],
        out_specs=[pl.BlockSpec((BLK, mid), lambda i: (i, 0))] +
                  [pl.BlockSpec((BLK, CH), lambda i: (i, 0))] * nch_out,
        out_shape=[jax.ShapeDtypeStruct((n_pad, mid), F32)] +
                  [jax.ShapeDtypeStruct((n_pad, CH), F32)] * nch_out,
    )(*s0, x_p, dis, W1, b1.reshape(1, -1), W2)


def _tc_layer2(s1, m1, dis, b2, W3):
    n_pad, mid = m1.shape
    out_dim = W3.shape[1]
    nb = n_pad // BLK
    nch_in = len(s1)
    nch_out = out_dim // CH

    def body(*refs):
        s_refs = refs[:nch_in]
        m1_ref, dis_ref, b2_ref, w3_ref = refs[nch_in:nch_in + 4]
        m2_ref = refs[nch_in + 4]
        g_refs = refs[nch_in + 5:]
        dis = dis_ref[...]
        h2 = jnp.maximum(dis * _combine(s_refs) +
                         (dis * dis) * m1_ref[...] + b2_ref[...], 0.0)
        m2 = _dot(h2, w3_ref[...])
        m2_ref[...] = m2
        g2 = dis * m2
        for ci in range(nch_out):
            g_refs[ci][...] = g2[:, ci * CH:(ci + 1) * CH]

    return pl.pallas_call(
        body,
        grid=(nb,),
        in_specs=[pl.BlockSpec((NC, BLK, CH), lambda i: (0, i, 0))] * nch_in + [
            pl.BlockSpec((BLK, mid), lambda i: (i, 0)),
            pl.BlockSpec((BLK, 1), lambda i: (i, 0)),
            pl.BlockSpec((1, mid), lambda i: (0, 0)),
            pl.BlockSpec((mid, out_dim), lambda i: (0, 0)),
        ],
        out_specs=[pl.BlockSpec((BLK, out_dim), lambda i: (i, 0))] +
                  [pl.BlockSpec((BLK, CH), lambda i: (i, 0))] * nch_out,
        out_shape=[jax.ShapeDtypeStruct((n_pad, out_dim), F32)] +
                  [jax.ShapeDtypeStruct((n_pad, CH), F32)] * nch_out,
    )(*s1, m1, dis, b2.reshape(1, -1), W3)


def _tc_final(s2, m2, dis, b3):
    n_pad, out_dim = m2.shape
    nb = n_pad // BLK
    nch_in = len(s2)

    def body(*refs):
        s_refs = refs[:nch_in]
        m2_ref, dis_ref, b3_ref, out_ref = refs[nch_in:]
        dis = dis_ref[...]
        out_ref[...] = jnp.maximum(dis * _combine(s_refs) +
                                   (dis * dis) * m2_ref[...] + b3_ref[...], 0.0)

    return pl.pallas_call(
        body,
        grid=(nb,),
        in_specs=[pl.BlockSpec((NC, BLK, CH), lambda i: (0, i, 0))] * nch_in + [
            pl.BlockSpec((BLK, out_dim), lambda i: (i, 0)),
            pl.BlockSpec((BLK, 1), lambda i: (i, 0)),
            pl.BlockSpec((1, out_dim), lambda i: (0, 0)),
        ],
        out_specs=pl.BlockSpec((BLK, out_dim), lambda i: (i, 0)),
        out_shape=jax.ShapeDtypeStruct((n_pad, out_dim), F32),
    )(*s2, m2, dis, b3.reshape(1, -1))


def kernel(x, edge_index, W1, b1, W2, b2, W3, b3):
    n, cin = x.shape
    e = edge_index.shape[1]

    r = _cdiv(e, NW * WIN)
    e_pad = NW * r * WIN
    n_pad = (_cdiv(n + 1, BLK)) * BLK

    src = edge_index[0].astype(jnp.int32)
    dst = edge_index[1].astype(jnp.int32)
    pad = jnp.full((e_pad - e,), n, jnp.int32)
    src3 = jnp.concatenate([src, pad]).reshape(NW, r, WIN)
    dst3 = jnp.concatenate([dst, pad]).reshape(NW, r, WIN)
    x_p = jnp.pad(x, ((0, n_pad - n), (0, 0)))

    hist = _sc_hist(n_pad, r)(dst3)
    dis, *g0 = _tc_prep(hist, x_p, n)
    s0 = _sc_scatter(len(g0), n_pad, r)(src3, dst3, *g0)
    m1, *g1 = _tc_layer1(s0, x_p, dis, W1, b1, W2)
    s1 = _sc_scatter(len(g1), n_pad, r)(src3, dst3, *g1)
    m2, *g2 = _tc_layer2(s1, m1, dis, b2, W3)
    s2 = _sc_scatter(len(g2), n_pad, r)(src3, dst3, *g2)
    out = _tc_final(s2, m2, dis, b3)
    return out[:n]


# speedup vs baseline: 9.3809x; 1.5735x over previous
import functools

import jax
import jax.numpy as jnp
from jax import lax
from jax.experimental import pallas as pl
from jax.experimental.pallas import tpu as pltpu
from jax.experimental.pallas import tpu_sc as plsc

F32 = jnp.float32

NC = 2
NS = 16
NW = NC * NS
WIN = 128
BLK = 512
CH = 16


def _cdiv(a, b):
    return (a + b - 1) // b


@functools.lru_cache(maxsize=None)
def _sc_hist(n_pad, r):
    mesh = plsc.VectorSubcoreMesh(core_axis_name="c", subcore_axis_name="s")
    rpw = n_pad // NS
    zrows = rpw // 4

    def body(dst_hbm, out_hbm, dst_v, ones_v, zbuf, acc, sem):
        c = lax.axis_index("c")
        s = lax.axis_index("s")
        wid = c * NS + s
        pltpu.sync_copy(dst_hbm.at[wid], dst_v)

        @pl.loop(0, WIN)
        def _(i):
            ones_v[i, pl.ds(0, 16)] = jnp.ones((16,), F32)

        @pl.loop(0, zrows)
        def _(i):
            zbuf[i, pl.ds(0, 16)] = jnp.zeros((16,), F32)

        for q in range(4):
            pltpu.sync_copy(zbuf, acc.at[pl.ds(s * rpw + q * zrows, zrows)])
        plsc.subcore_barrier()

        @pl.loop(0, r)
        def _(j):
            pltpu.async_copy(ones_v, acc.at[dst_v.at[j]], sem, add=True).wait()

        plsc.subcore_barrier()
        pltpu.sync_copy(acc.at[pl.ds(s * rpw, rpw)],
                        out_hbm.at[c, pl.ds(s * rpw, rpw)])

    return pl.kernel(
        body,
        mesh=mesh,
        compiler_params=pltpu.CompilerParams(use_tc_tiling_on_sc=False),
        out_type=jax.ShapeDtypeStruct((NC, n_pad, 16), F32),
        scratch_types=[
            pltpu.VMEM((r, WIN), jnp.int32),
            pltpu.VMEM((WIN, 16), F32),
            pltpu.VMEM((zrows, 16), F32),
            pltpu.VMEM_SHARED((n_pad, 16), F32),
            pltpu.SemaphoreType.DMA,
        ],
    )


@functools.lru_cache(maxsize=None)
def _sc_scatter(n_chunks, n_pad, r):
    mesh = plsc.VectorSubcoreMesh(core_axis_name="c", subcore_axis_name="s")
    rpw = n_pad // NS
    zrows = rpw // 4

    nbuf = 4
    assert r % nbuf == 0

    def body(src_hbm, dst_hbm, *rest):
        g_refs = rest[:n_chunks]
        out_refs = rest[n_chunks:2 * n_chunks]
        rest = rest[2 * n_chunks:]
        src_v, dst_v = rest[0], rest[1]
        bufs = rest[2:2 + nbuf]
        zbuf = rest[2 + nbuf]
        acc = rest[3 + nbuf]
        sems_g = rest[4 + nbuf:4 + 2 * nbuf]
        sems_s = rest[4 + 2 * nbuf:4 + 3 * nbuf]
        c = lax.axis_index("c")
        s = lax.axis_index("s")
        wid = c * NS + s
        pltpu.sync_copy(src_hbm.at[wid], src_v)
        pltpu.sync_copy(dst_hbm.at[wid], dst_v)

        @pl.loop(0, zrows)
        def _(i):
            for h in range(CH // 16):
                zbuf[i, pl.ds(h * 16, 16)] = jnp.zeros((16,), F32)

        for ci in range(n_chunks):
            for q in range(4):
                pltpu.sync_copy(zbuf, acc.at[pl.ds(s * rpw + q * zrows, zrows)])
            plsc.subcore_barrier()

            @pl.loop(0, r, step=nbuf)
            def _(j):
                gathers = [
                    pltpu.async_copy(g_refs[ci].at[src_v.at[j + b]], bufs[b],
                                     sems_g[b])
                    for b in range(nbuf)
                ]
                scatters = []
                for b in range(nbuf):
                    gathers[b].wait()
                    scatters.append(
                        pltpu.async_copy(bufs[b], acc.at[dst_v.at[j + b]],
                                         sems_s[b], add=True))
                for b in range(nbuf):
                    scatters[b].wait()

            plsc.subcore_barrier()
            pltpu.sync_copy(acc.at[pl.ds(s * rpw, rpw)],
                            out_refs[ci].at[c, pl.ds(s * rpw, rpw)])
            if ci + 1 < n_chunks:
                plsc.subcore_barrier()

    return pl.kernel(
        body,
        mesh=mesh,
        compiler_params=pltpu.CompilerParams(use_tc_tiling_on_sc=False),
        out_type=tuple(jax.ShapeDtypeStruct((NC, n_pad, CH), F32)
                       for _ in range(n_chunks)),
        scratch_types=[
            pltpu.VMEM((r, WIN), jnp.int32),
            pltpu.VMEM((r, WIN), jnp.int32),
        ] + [pltpu.VMEM((WIN, CH), F32)] * nbuf + [
            pltpu.VMEM((zrows, CH), F32),
            pltpu.VMEM_SHARED((n_pad, CH), F32),
        ] + [pltpu.SemaphoreType.DMA] * (2 * nbuf),
    )


def _dot(a, b):
    return lax.dot_general(a, b, (((1,), (0,)), ((), ())),
                           precision=lax.Precision.HIGHEST,
                           preferred_element_type=F32)


def _tc_prep(hist, x_p, n_real):
    n_pad, cin = x_p.shape
    nb = n_pad // BLK
    nch = cin // CH

    def body(hist_ref, x_ref, dis_ref, *g_refs):
        i = pl.program_id(0)
        deg = hist_ref[0] + hist_ref[1] + 1.0
        row = i * BLK + lax.broadcasted_iota(jnp.int32, (BLK, 16), 0)
        dis = jnp.where(row < n_real, lax.rsqrt(deg), 0.0)
        dis_col = dis[:, 0:1]
        dis_ref[...] = dis_col
        xs = x_ref[...] * dis_col
        for ci in range(nch):
            g_refs[ci][...] = xs[:, ci * CH:(ci + 1) * CH]

    return pl.pallas_call(
        body,
        grid=(nb,),
        in_specs=[
            pl.BlockSpec((NC, BLK, 16), lambda i: (0, i, 0)),
            pl.BlockSpec((BLK, cin), lambda i: (i, 0)),
        ],
        out_specs=[pl.BlockSpec((BLK, 1), lambda i: (i, 0))] +
                  [pl.BlockSpec((BLK, CH), lambda i: (i, 0))] * nch,
        out_shape=[jax.ShapeDtypeStruct((n_pad, 1), F32)] +
                  [jax.ShapeDtypeStruct((n_pad, CH), F32)] * nch,
    )(hist, x_p)


def _combine(s_refs):
    return jnp.concatenate([ref[0] + ref[1] for ref in s_refs], axis=1)


def _tc_layer1(s0, x_p, dis, W1, b1, W2):
    n_pad, cin = x_p.shape
    hid = W1.shape[1]
    mid = W2.shape[1]
    nb = n_pad // BLK
    nch_in = len(s0)
    nch_out = mid // CH

    def body(*refs):
        s_refs = refs[:nch_in]
        x_ref, dis_ref, w1_ref, b1_ref, w2_ref = refs[nch_in:nch_in + 5]
        m1_ref = refs[nch_in + 5]
        g_refs = refs[nch_in + 6:]
        dis = dis_ref[...]
        p0 = dis * _combine(s_refs) + (dis * dis) * x_ref[...]
        h1 = jnp.maximum(_dot(p0, w1_ref[...]) + b1_ref[...], 0.0)
        m1 = _dot(h1, w2_ref[...])
        m1_ref[...] = m1
        g1 = dis * m1
        for ci in range(nch_out):
            g_refs[ci][...] = g1[:, ci * CH:(ci + 1) * CH]

    return pl.pallas_call(
        body,
        grid=(nb,),
        in_specs=[pl.BlockSpec((NC, BLK, CH), lambda i: (0, i, 0))] * nch_in + [
            pl.BlockSpec((BLK, cin), lambda i: (i, 0)),
            pl.BlockSpec((BLK, 1), lambda i: (i, 0)),
            pl.BlockSpec((cin, hid), lambda i: (0, 0)),
            pl.BlockSpec((1, hid), lambda i: (0, 0)),
            pl.BlockSpec((hid, mid), lambda i: (0, 0)),
        ],
        out_specs=[pl.BlockSpec((BLK, mid), lambda i: (i, 0))] +
                  [pl.BlockSpec((BLK, CH), lambda i: (i, 0))] * nch_out,
        out_shape=[jax.ShapeDtypeStruct((n_pad, mid), F32)] +
                  [jax.ShapeDtypeStruct((n_pad, CH), F32)] * nch_out,
    )(*s0, x_p, dis, W1, b1.reshape(1, -1), W2)


def _tc_layer2(s1, m1, dis, b2, W3):
    n_pad, mid = m1.shape
    out_dim = W3.shape[1]
    nb = n_pad // BLK
    nch_in = len(s1)
    nch_out = out_dim // CH

    def body(*refs):
        s_refs = refs[:nch_in]
        m1_ref, dis_ref, b2_ref, w3_ref = refs[nch_in:nch_in + 4]
        m2_ref = refs[nch_in + 4]
        g_refs = refs[nch_in + 5:]
        dis = dis_ref[...]
        h2 = jnp.maximum(dis * _combine(s_refs) +
                         (dis * dis) * m1_ref[...] + b2_ref[...], 0.0)
        m2 = _dot(h2, w3_ref[...])
        m2_ref[...] = m2
        g2 = dis * m2
        for ci in range(nch_out):
            g_refs[ci][...] = g2[:, ci * CH:(ci + 1) * CH]

    return pl.pallas_call(
        body,
        grid=(nb,),
        in_specs=[pl.BlockSpec((NC, BLK, CH), lambda i: (0, i, 0))] * nch_in + [
            pl.BlockSpec((BLK, mid), lambda i: (i, 0)),
            pl.BlockSpec((BLK, 1), lambda i: (i, 0)),
            pl.BlockSpec((1, mid), lambda i: (0, 0)),
            pl.BlockSpec((mid, out_dim), lambda i: (0, 0)),
        ],
        out_specs=[pl.BlockSpec((BLK, out_dim), lambda i: (i, 0))] +
                  [pl.BlockSpec((BLK, CH), lambda i: (i, 0))] * nch_out,
        out_shape=[jax.ShapeDtypeStruct((n_pad, out_dim), F32)] +
                  [jax.ShapeDtypeStruct((n_pad, CH), F32)] * nch_out,
    )(*s1, m1, dis, b2.reshape(1, -1), W3)


def _tc_final(s2, m2, dis, b3):
    n_pad, out_dim = m2.shape
    nb = n_pad // BLK
    nch_in = len(s2)

    def body(*refs):
        s_refs = refs[:nch_in]
        m2_ref, dis_ref, b3_ref, out_ref = refs[nch_in:]
        dis = dis_ref[...]
        out_ref[...] = jnp.maximum(dis * _combine(s_refs) +
                                   (dis * dis) * m2_ref[...] + b3_ref[...], 0.0)

    return pl.pallas_call(
        body,
        grid=(nb,),
        in_specs=[pl.BlockSpec((NC, BLK, CH), lambda i: (0, i, 0))] * nch_in + [
            pl.BlockSpec((BLK, out_dim), lambda i: (i, 0)),
            pl.BlockSpec((BLK, 1), lambda i: (i, 0)),
            pl.BlockSpec((1, out_dim), lambda i: (0, 0)),
        ],
        out_specs=pl.BlockSpec((BLK, out_dim), lambda i: (i, 0)),
        out_shape=jax.ShapeDtypeStruct((n_pad, out_dim), F32),
    )(*s2, m2, dis, b3.reshape(1, -1))


def kernel(x, edge_index, W1, b1, W2, b2, W3, b3):
    n, cin = x.shape
    e = edge_index.shape[1]

    r = _cdiv(e, NW * WIN)
    e_pad = NW * r * WIN
    n_pad = (_cdiv(n + 1, BLK)) * BLK

    src = edge_index[0].astype(jnp.int32)
    dst = edge_index[1].astype(jnp.int32)
    pad = jnp.full((e_pad - e,), n, jnp.int32)
    src3 = jnp.concatenate([src, pad]).reshape(NW, r, WIN)
    dst3 = jnp.concatenate([dst, pad]).reshape(NW, r, WIN)
    x_p = jnp.pad(x, ((0, n_pad - n), (0, 0)))

    hist = _sc_hist(n_pad, r)(dst3)
    dis, *g0 = _tc_prep(hist, x_p, n)
    s0 = _sc_scatter(len(g0), n_pad, r)(src3, dst3, *g0)
    m1, *g1 = _tc_layer1(s0, x_p, dis, W1, b1, W2)
    s1 = _sc_scatter(len(g1), n_pad, r)(src3, dst3, *g1)
    m2, *g2 = _tc_layer2(s1, m1, dis, b2, W3)
    s2 = _sc_scatter(len(g2), n_pad, r)(src3, dst3, *g2)
    out = _tc_final(s2, m2, dis, b3)
    return out[:n]


# speedup vs baseline: 12.7743x; 1.3617x over previous
import functools

import jax
import jax.numpy as jnp
from jax import lax
from jax.experimental import pallas as pl
from jax.experimental.pallas import tpu as pltpu
from jax.experimental.pallas import tpu_sc as plsc

F32 = jnp.float32

NC = 2
NS = 16
WIN = 128
BLK = 512
CH = 16
NBUF = 4


def _cdiv(a, b):
    return (a + b - 1) // b


@functools.lru_cache(maxsize=None)
def _sc_hist(n_pad, r2):
    mesh = plsc.VectorSubcoreMesh(core_axis_name="c", subcore_axis_name="s")
    rpw = n_pad // NS
    zrows = rpw // 4
    rh = r2 // 2

    def body(dst_hbm, out_hbm, dst_v, ones_v, zbuf, acc, sem):
        c = lax.axis_index("c")
        s = lax.axis_index("s")
        pltpu.sync_copy(dst_hbm.at[s, pl.ds(c * rh, rh)], dst_v)

        @pl.loop(0, WIN)
        def _(i):
            ones_v[i, pl.ds(0, CH)] = jnp.ones((CH,), F32)

        @pl.loop(0, zrows)
        def _(i):
            zbuf[i, pl.ds(0, CH)] = jnp.zeros((CH,), F32)

        for q in range(4):
            pltpu.sync_copy(zbuf, acc.at[pl.ds(s * rpw + q * zrows, zrows)])
        plsc.subcore_barrier()

        @pl.loop(0, rh)
        def _(j):
            pltpu.async_copy(ones_v, acc.at[dst_v.at[j]], sem, add=True).wait()

        plsc.subcore_barrier()
        pltpu.sync_copy(acc.at[pl.ds(s * rpw, rpw)],
                        out_hbm.at[c, pl.ds(s * rpw, rpw)])

    return pl.kernel(
        body,
        mesh=mesh,
        compiler_params=pltpu.CompilerParams(use_tc_tiling_on_sc=False),
        out_type=jax.ShapeDtypeStruct((NC, n_pad, CH), F32),
        scratch_types=[
            pltpu.VMEM((rh, WIN), jnp.int32),
            pltpu.VMEM((WIN, CH), F32),
            pltpu.VMEM((zrows, CH), F32),
            pltpu.VMEM_SHARED((n_pad, CH), F32),
            pltpu.SemaphoreType.DMA,
        ],
    )


@functools.lru_cache(maxsize=None)
def _sc_scatter(n_chunks, n_pad, r2):
    mesh = plsc.VectorSubcoreMesh(core_axis_name="c", subcore_axis_name="s")
    rpw = n_pad // NS
    zrows = rpw // 4
    rh = r2 // 2
    cpc = n_chunks // NC
    assert rh % NBUF == 0

    def body(src8_hbm, dst_hbm, t8_hbm, out_hbm, *rest):
        src_v, dst_v = rest[0], rest[1]
        bufs = rest[2:2 + NBUF]
        zbuf = rest[2 + NBUF]
        acc = rest[3 + NBUF]
        sems_g = rest[4 + NBUF:4 + 2 * NBUF]
        sems_s = rest[4 + 2 * NBUF:4 + 3 * NBUF]
        c = lax.axis_index("c")
        s = lax.axis_index("s")

        @pl.loop(0, zrows)
        def _(i):
            zbuf[i, pl.ds(0, CH)] = jnp.zeros((CH,), F32)

        for ci_l in range(cpc):
            ci = c * cpc + ci_l
            for q in range(4):
                pltpu.sync_copy(zbuf, acc.at[pl.ds(s * rpw + q * zrows, zrows)])
            plsc.subcore_barrier()

            for half in range(2):
                pltpu.sync_copy(src8_hbm.at[ci, s, pl.ds(half * rh, rh)], src_v)
                pltpu.sync_copy(dst_hbm.at[s, pl.ds(half * rh, rh)], dst_v)

                @pl.loop(0, rh, step=NBUF)
                def _(j):
                    @pl.when(j > 0)
                    def _():
                        for b in range(NBUF):
                            pltpu.make_async_copy(
                                bufs[b], acc.at[dst_v.at[j + b]],
                                sems_s[b]).wait()
                    gathers = [
                        pltpu.async_copy(t8_hbm.at[src_v.at[j + b]], bufs[b],
                                         sems_g[b])
                        for b in range(NBUF)
                    ]
                    for b in range(NBUF):
                        gathers[b].wait()
                        pltpu.async_copy(bufs[b], acc.at[dst_v.at[j + b]],
                                         sems_s[b], add=True)

                for b in range(NBUF):
                    pltpu.make_async_copy(bufs[b], acc.at[dst_v.at[rh - NBUF + b]],
                                          sems_s[b]).wait()

            plsc.subcore_barrier()
            pltpu.sync_copy(acc.at[pl.ds(s * rpw, rpw)],
                            out_hbm.at[ci, pl.ds(s * rpw, rpw)])
            if ci_l + 1 < cpc:
                plsc.subcore_barrier()

    return pl.kernel(
        body,
        mesh=mesh,
        compiler_params=pltpu.CompilerParams(use_tc_tiling_on_sc=False),
        out_type=jax.ShapeDtypeStruct((n_chunks, n_pad, CH), F32),
        scratch_types=[
            pltpu.VMEM((rh, WIN), jnp.int32),
            pltpu.VMEM((rh, WIN), jnp.int32),
        ] + [pltpu.VMEM((WIN, CH), F32)] * NBUF + [
            pltpu.VMEM((zrows, CH), F32),
            pltpu.VMEM_SHARED((n_pad, CH), F32),
        ] + [pltpu.SemaphoreType.DMA] * (2 * NBUF),
    )


def _dot(a, b):
    return lax.dot_general(a, b, (((1,), (0,)), ((), ())),
                           precision=lax.Precision.HIGHEST,
                           preferred_element_type=F32)


def _tc_prep(hist, x_p, n_real):
    n_pad, cin = x_p.shape
    nb = n_pad // BLK

    def body(hist_ref, x_ref, dis_ref, t_ref):
        i = pl.program_id(0)
        deg = hist_ref[0] + hist_ref[1] + 1.0
        row = i * BLK + lax.broadcasted_iota(jnp.int32, (BLK, CH), 0)
        dis = jnp.where(row < n_real, lax.rsqrt(deg), 0.0)
        dis_col = dis[:, 0:1]
        dis_ref[...] = dis_col
        t_ref[...] = x_ref[...] * dis_col

    return pl.pallas_call(
        body,
        grid=(nb,),
        in_specs=[
            pl.BlockSpec((NC, BLK, CH), lambda i: (0, i, 0)),
            pl.BlockSpec((BLK, cin), lambda i: (i, 0)),
        ],
        out_specs=[pl.BlockSpec((BLK, 1), lambda i: (i, 0)),
                   pl.BlockSpec((BLK, cin), lambda i: (i, 0))],
        out_shape=[jax.ShapeDtypeStruct((n_pad, 1), F32),
                   jax.ShapeDtypeStruct((n_pad, cin), F32)],
    )(hist, x_p)


def _tc_layer(s_full, t_prev, dis, b, W_next):
    n_pad, fin = t_prev.shape
    fout = W_next.shape[1]
    nb = n_pad // BLK

    def body(s_ref, t_ref, dis_ref, b_ref, w_ref, out_ref):
        dis = dis_ref[...]
        h = jnp.maximum(dis * (s_ref[...] + t_ref[...]) + b_ref[...], 0.0)
        out_ref[...] = dis * _dot(h, w_ref[...])

    return pl.pallas_call(
        body,
        grid=(nb,),
        in_specs=[
            pl.BlockSpec((BLK, fin), lambda i: (i, 0)),
            pl.BlockSpec((BLK, fin), lambda i: (i, 0)),
            pl.BlockSpec((BLK, 1), lambda i: (i, 0)),
            pl.BlockSpec((1, fin), lambda i: (0, 0)),
            pl.BlockSpec((fin, fout), lambda i: (0, 0)),
        ],
        out_specs=pl.BlockSpec((BLK, fout), lambda i: (i, 0)),
        out_shape=jax.ShapeDtypeStruct((n_pad, fout), F32),
    )(s_full, t_prev, dis, b.reshape(1, -1), W_next)


def _tc_layer1(s_full, t0, dis, W1, b1, W2):
    n_pad, cin = t0.shape
    hid = W1.shape[1]
    mid = W2.shape[1]
    nb = n_pad // BLK

    def body(s_ref, t_ref, dis_ref, w1_ref, b1_ref, w2_ref, out_ref):
        dis = dis_ref[...]
        p0 = dis * (s_ref[...] + t_ref[...])
        h1 = jnp.maximum(_dot(p0, w1_ref[...]) + b1_ref[...], 0.0)
        out_ref[...] = dis * _dot(h1, w2_ref[...])

    return pl.pallas_call(
        body,
        grid=(nb,),
        in_specs=[
            pl.BlockSpec((BLK, cin), lambda i: (i, 0)),
            pl.BlockSpec((BLK, cin), lambda i: (i, 0)),
            pl.BlockSpec((BLK, 1), lambda i: (i, 0)),
            pl.BlockSpec((cin, hid), lambda i: (0, 0)),
            pl.BlockSpec((1, hid), lambda i: (0, 0)),
            pl.BlockSpec((hid, mid), lambda i: (0, 0)),
        ],
        out_specs=pl.BlockSpec((BLK, mid), lambda i: (i, 0)),
        out_shape=jax.ShapeDtypeStruct((n_pad, mid), F32),
    )(s_full, t0, dis, W1, b1.reshape(1, -1), W2)


def _tc_final(s_full, t2, dis, b3):
    n_pad, fout = t2.shape
    nb = n_pad // BLK

    def body(s_ref, t_ref, dis_ref, b_ref, out_ref):
        dis = dis_ref[...]
        out_ref[...] = jnp.maximum(dis * (s_ref[...] + t_ref[...]) + b_ref[...],
                                   0.0)

    return pl.pallas_call(
        body,
        grid=(nb,),
        in_specs=[
            pl.BlockSpec((BLK, fout), lambda i: (i, 0)),
            pl.BlockSpec((BLK, fout), lambda i: (i, 0)),
            pl.BlockSpec((BLK, 1), lambda i: (i, 0)),
            pl.BlockSpec((1, fout), lambda i: (0, 0)),
        ],
        out_specs=pl.BlockSpec((BLK, fout), lambda i: (i, 0)),
        out_shape=jax.ShapeDtypeStruct((n_pad, fout), F32),
    )(s_full, t2, dis, b3.reshape(1, -1))


def kernel(x, edge_index, W1, b1, W2, b2, W3, b3):
    n, cin = x.shape
    e = edge_index.shape[1]

    r2 = _cdiv(e, NS * WIN)
    r2 += r2 % 2
    e_pad = NS * r2 * WIN
    n_pad = (_cdiv(n + 1, BLK)) * BLK

    src = edge_index[0].astype(jnp.int32)
    dst = edge_index[1].astype(jnp.int32)
    pad = jnp.full((e_pad - e,), n, jnp.int32)
    src_p = jnp.concatenate([src, pad])
    dst2 = jnp.concatenate([dst, pad]).reshape(NS, r2, WIN)
    nch = cin // CH
    src8 = (src_p * 8)[None, :] + jnp.arange(nch, dtype=jnp.int32)[:, None]
    src8 = src8.reshape(nch, NS, r2, WIN)
    nch3 = (W3.shape[1]) // CH
    src4 = (src_p * nch3)[None, :] + jnp.arange(nch3, dtype=jnp.int32)[:, None]
    src4 = src4.reshape(nch3, NS, r2, WIN)
    x_p = jnp.pad(x, ((0, n_pad - n), (0, 0)))

    def scat(t, n_chunks, srcb_slabs):
        t8 = t.reshape(n_pad * (t.shape[1] // CH), CH)
        s = _sc_scatter(n_chunks, n_pad, r2)(srcb_slabs, dst2, t8)
        return s.transpose(1, 0, 2).reshape(n_pad, n_chunks * CH)

    hist = _sc_hist(n_pad, r2)(dst2)
    dis, t0 = _tc_prep(hist, x_p, n)
    s0 = scat(t0, nch, src8)
    t1 = _tc_layer1(s0, t0, dis, W1, b1, W2)
    s1 = scat(t1, nch, src8)
    t2 = _tc_layer(s1, t1, dis, b2, W3)
    s2 = scat(t2, nch3, src4)
    out = _tc_final(s2, t2, dis, b3)
    return out[:n]


# speedup vs baseline: 13.9713x; 1.0937x over previous
import functools

import jax
import jax.numpy as jnp
from jax import lax
from jax.experimental import pallas as pl
from jax.experimental.pallas import tpu as pltpu
from jax.experimental.pallas import tpu_sc as plsc

F32 = jnp.float32

NC = 2
NS = 16
WIN = 128
BLK = 512
CH = 32
HCH = 16
NBUF = 2
PIECE = 56


def _cdiv(a, b):
    return (a + b - 1) // b


@functools.lru_cache(maxsize=None)
def _sc_hist(n_pad, r2):
    mesh = plsc.VectorSubcoreMesh(core_axis_name="c", subcore_axis_name="s")
    rpw = n_pad // NS
    zrows = rpw // 4
    rh = r2 // 2

    def body(dst_hbm, out_hbm, dst_v, ones_v, zbuf, acc, sem):
        c = lax.axis_index("c")
        s = lax.axis_index("s")
        pltpu.sync_copy(dst_hbm.at[s, pl.ds(c * rh, rh)], dst_v)

        @pl.loop(0, WIN)
        def _(i):
            ones_v[i, pl.ds(0, HCH)] = jnp.ones((HCH,), F32)

        @pl.loop(0, zrows)
        def _(i):
            zbuf[i, pl.ds(0, HCH)] = jnp.zeros((HCH,), F32)

        for q in range(4):
            pltpu.sync_copy(zbuf, acc.at[pl.ds(s * rpw + q * zrows, zrows)])
        plsc.subcore_barrier()

        @pl.loop(0, rh)
        def _(j):
            pltpu.async_copy(ones_v, acc.at[dst_v.at[j]], sem, add=True).wait()

        plsc.subcore_barrier()
        pltpu.sync_copy(acc.at[pl.ds(s * rpw, rpw)],
                        out_hbm.at[c, pl.ds(s * rpw, rpw)])

    return pl.kernel(
        body,
        mesh=mesh,
        compiler_params=pltpu.CompilerParams(use_tc_tiling_on_sc=False),
        out_type=jax.ShapeDtypeStruct((NC, n_pad, HCH), F32),
        scratch_types=[
            pltpu.VMEM((rh, WIN), jnp.int32),
            pltpu.VMEM((WIN, HCH), F32),
            pltpu.VMEM((zrows, HCH), F32),
            pltpu.VMEM_SHARED((n_pad, HCH), F32),
            pltpu.SemaphoreType.DMA,
        ],
    )


@functools.lru_cache(maxsize=None)
def _sc_scatter(n_chunks, n_pad, r2):
    mesh = plsc.VectorSubcoreMesh(core_axis_name="c", subcore_axis_name="s")
    rpw = n_pad // NS
    zrows = rpw // 16
    cpc = n_chunks // NC
    n_pieces = r2 // PIECE
    assert r2 % PIECE == 0 and PIECE % NBUF == 0

    def body(src8_hbm, dst_hbm, t8_hbm, out_hbm, *rest):
        src_v, dst_v = rest[0], rest[1]
        bufs = rest[2:2 + NBUF]
        zbuf = rest[2 + NBUF]
        acc = rest[3 + NBUF]
        sems_g = rest[4 + NBUF:4 + 2 * NBUF]
        sems_s = rest[4 + 2 * NBUF:4 + 3 * NBUF]
        c = lax.axis_index("c")
        s = lax.axis_index("s")

        @pl.loop(0, zrows)
        def _(i):
            zbuf[i, pl.ds(0, CH)] = jnp.zeros((CH,), F32)

        for ci_l in range(cpc):
            ci = c * cpc + ci_l
            for q in range(16):
                pltpu.sync_copy(zbuf, acc.at[pl.ds(s * rpw + q * zrows, zrows)])
            plsc.subcore_barrier()

            for piece in range(n_pieces):
                pltpu.sync_copy(src8_hbm.at[ci, s, pl.ds(piece * PIECE, PIECE)],
                                src_v)
                pltpu.sync_copy(dst_hbm.at[s, pl.ds(piece * PIECE, PIECE)],
                                dst_v)

                @pl.loop(0, PIECE, step=NBUF)
                def _(j):
                    @pl.when(j > 0)
                    def _():
                        for b in range(NBUF):
                            pltpu.make_async_copy(
                                bufs[b], acc.at[dst_v.at[j + b]],
                                sems_s[b]).wait()
                    gathers = [
                        pltpu.async_copy(t8_hbm.at[src_v.at[j + b]], bufs[b],
                                         sems_g[b])
                        for b in range(NBUF)
                    ]
                    for b in range(NBUF):
                        gathers[b].wait()
                        pltpu.async_copy(bufs[b], acc.at[dst_v.at[j + b]],
                                         sems_s[b], add=True)

                for b in range(NBUF):
                    pltpu.make_async_copy(
                        bufs[b], acc.at[dst_v.at[PIECE - NBUF + b]],
                        sems_s[b]).wait()

            plsc.subcore_barrier()
            pltpu.sync_copy(acc.at[pl.ds(s * rpw, rpw)],
                            out_hbm.at[ci, pl.ds(s * rpw, rpw)])
            if ci_l + 1 < cpc:
                plsc.subcore_barrier()

    return pl.kernel(
        body,
        mesh=mesh,
        compiler_params=pltpu.CompilerParams(use_tc_tiling_on_sc=False),
        out_type=jax.ShapeDtypeStruct((n_chunks, n_pad, CH), F32),
        scratch_types=[
            pltpu.VMEM((PIECE, WIN), jnp.int32),
            pltpu.VMEM((PIECE, WIN), jnp.int32),
        ] + [pltpu.VMEM((WIN, CH), F32)] * NBUF + [
            pltpu.VMEM((zrows, CH), F32),
            pltpu.VMEM_SHARED((n_pad, CH), F32),
        ] + [pltpu.SemaphoreType.DMA] * (2 * NBUF),
    )


def _dot(a, b):
    return lax.dot_general(a, b, (((1,), (0,)), ((), ())),
                           precision=lax.Precision.HIGHEST,
                           preferred_element_type=F32)


def _tc_prep(hist, x_p, n_real):
    n_pad, cin = x_p.shape
    nb = n_pad // BLK

    def body(hist_ref, x_ref, dis_ref, t_ref):
        i = pl.program_id(0)
        deg = hist_ref[0] + hist_ref[1] + 1.0
        row = i * BLK + lax.broadcasted_iota(jnp.int32, (BLK, HCH), 0)
        dis = jnp.where(row < n_real, lax.rsqrt(deg), 0.0)
        dis_col = dis[:, 0:1]
        dis_ref[...] = dis_col
        t_ref[...] = x_ref[...] * dis_col

    return pl.pallas_call(
        body,
        grid=(nb,),
        in_specs=[
            pl.BlockSpec((NC, BLK, HCH), lambda i: (0, i, 0)),
            pl.BlockSpec((BLK, cin), lambda i: (i, 0)),
        ],
        out_specs=[pl.BlockSpec((BLK, 1), lambda i: (i, 0)),
                   pl.BlockSpec((BLK, cin), lambda i: (i, 0))],
        out_shape=[jax.ShapeDtypeStruct((n_pad, 1), F32),
                   jax.ShapeDtypeStruct((n_pad, cin), F32)],
    )(hist, x_p)


def _tc_layer(s_full, t_prev, dis, b, W_next):
    n_pad, fin = t_prev.shape
    fout = W_next.shape[1]
    nb = n_pad // BLK

    def body(s_ref, t_ref, dis_ref, b_ref, w_ref, out_ref):
        dis = dis_ref[...]
        h = jnp.maximum(dis * (s_ref[...] + t_ref[...]) + b_ref[...], 0.0)
        out_ref[...] = dis * _dot(h, w_ref[...])

    return pl.pallas_call(
        body,
        grid=(nb,),
        in_specs=[
            pl.BlockSpec((BLK, fin), lambda i: (i, 0)),
            pl.BlockSpec((BLK, fin), lambda i: (i, 0)),
            pl.BlockSpec((BLK, 1), lambda i: (i, 0)),
            pl.BlockSpec((1, fin), lambda i: (0, 0)),
            pl.BlockSpec((fin, fout), lambda i: (0, 0)),
        ],
        out_specs=pl.BlockSpec((BLK, fout), lambda i: (i, 0)),
        out_shape=jax.ShapeDtypeStruct((n_pad, fout), F32),
    )(s_full, t_prev, dis, b.reshape(1, -1), W_next)


def _tc_layer1(s_full, t0, dis, W1, b1, W2):
    n_pad, cin = t0.shape
    hid = W1.shape[1]
    mid = W2.shape[1]
    nb = n_pad // BLK

    def body(s_ref, t_ref, dis_ref, w1_ref, b1_ref, w2_ref, out_ref):
        dis = dis_ref[...]
        p0 = dis * (s_ref[...] + t_ref[...])
        h1 = jnp.maximum(_dot(p0, w1_ref[...]) + b1_ref[...], 0.0)
        out_ref[...] = dis * _dot(h1, w2_ref[...])

    return pl.pallas_call(
        body,
        grid=(nb,),
        in_specs=[
            pl.BlockSpec((BLK, cin), lambda i: (i, 0)),
            pl.BlockSpec((BLK, cin), lambda i: (i, 0)),
            pl.BlockSpec((BLK, 1), lambda i: (i, 0)),
            pl.BlockSpec((cin, hid), lambda i: (0, 0)),
            pl.BlockSpec((1, hid), lambda i: (0, 0)),
            pl.BlockSpec((hid, mid), lambda i: (0, 0)),
        ],
        out_specs=pl.BlockSpec((BLK, mid), lambda i: (i, 0)),
        out_shape=jax.ShapeDtypeStruct((n_pad, mid), F32),
    )(s_full, t0, dis, W1, b1.reshape(1, -1), W2)


def _tc_final(s_full, t2, dis, b3):
    n_pad, fout = t2.shape
    nb = n_pad // BLK

    def body(s_ref, t_ref, dis_ref, b_ref, out_ref):
        dis = dis_ref[...]
        out_ref[...] = jnp.maximum(dis * (s_ref[...] + t_ref[...]) + b_ref[...],
                                   0.0)

    return pl.pallas_call(
        body,
        grid=(nb,),
        in_specs=[
            pl.BlockSpec((BLK, fout), lambda i: (i, 0)),
            pl.BlockSpec((BLK, fout), lambda i: (i, 0)),
            pl.BlockSpec((BLK, 1), lambda i: (i, 0)),
            pl.BlockSpec((1, fout), lambda i: (0, 0)),
        ],
        out_specs=pl.BlockSpec((BLK, fout), lambda i: (i, 0)),
        out_shape=jax.ShapeDtypeStruct((n_pad, fout), F32),
    )(s_full, t2, dis, b3.reshape(1, -1))


def kernel(x, edge_index, W1, b1, W2, b2, W3, b3):
    n, cin = x.shape
    e = edge_index.shape[1]

    r2 = _cdiv(_cdiv(e, NS * WIN), PIECE) * PIECE
    e_pad = NS * r2 * WIN
    n_pad = (_cdiv(n + 1, BLK)) * BLK

    src = edge_index[0].astype(jnp.int32)
    dst = edge_index[1].astype(jnp.int32)
    pad = jnp.full((e_pad - e,), n, jnp.int32)
    src_p = jnp.concatenate([src, pad])
    dst2 = jnp.concatenate([dst, pad]).reshape(NS, r2, WIN)
    nch = cin // CH
    src8 = (src_p * nch)[None, :] + jnp.arange(nch, dtype=jnp.int32)[:, None]
    src8 = src8.reshape(nch, NS, r2, WIN)
    nch3 = (W3.shape[1]) // CH
    src4 = (src_p * nch3)[None, :] + jnp.arange(nch3, dtype=jnp.int32)[:, None]
    src4 = src4.reshape(nch3, NS, r2, WIN)
    x_p = jnp.pad(x, ((0, n_pad - n), (0, 0)))

    def scat(t, n_chunks, srcb_slabs):
        t8 = t.reshape(n_pad * (t.shape[1] // CH), CH)
        s = _sc_scatter(n_chunks, n_pad, r2)(srcb_slabs, dst2, t8)
        return s.transpose(1, 0, 2).reshape(n_pad, n_chunks * CH)

    hist = _sc_hist(n_pad, r2)(dst2)
    dis, t0 = _tc_prep(hist, x_p, n)
    s0 = scat(t0, nch, src8)
    t1 = _tc_layer1(s0, t0, dis, W1, b1, W2)
    s1 = scat(t1, nch, src8)
    t2 = _tc_layer(s1, t1, dis, b2, W3)
    s2 = scat(t2, nch3, src4)
    out = _tc_final(s2, t2, dis, b3)
    return out[:n]


# speedup vs baseline: 15.5580x; 1.1136x over previous
import functools

import jax
import jax.numpy as jnp
from jax import lax
from jax.experimental import pallas as pl
from jax.experimental.pallas import tpu as pltpu
from jax.experimental.pallas import tpu_sc as plsc

F32 = jnp.float32

NC = 2
NS = 16
WIN = 128
BLK = 512
CH = 32
HCH = 16
NBUF = 2
PIECE = 56


def _cdiv(a, b):
    return (a + b - 1) // b


@functools.lru_cache(maxsize=None)
def _sc_hist(n_pad, r2):
    mesh = plsc.VectorSubcoreMesh(core_axis_name="c", subcore_axis_name="s")
    rpw = n_pad // NS
    zrows = rpw // 4
    rh = r2 // 2

    def body(dst_hbm, out_hbm, dst_v, ones_v, zbuf, acc, sem):
        c = lax.axis_index("c")
        s = lax.axis_index("s")
        pltpu.sync_copy(dst_hbm.at[s, pl.ds(c * rh, rh)], dst_v)

        @pl.loop(0, WIN)
        def _(i):
            ones_v[i, pl.ds(0, HCH)] = jnp.ones((HCH,), F32)

        @pl.loop(0, zrows)
        def _(i):
            zbuf[i, pl.ds(0, HCH)] = jnp.zeros((HCH,), F32)

        for q in range(4):
            pltpu.sync_copy(zbuf, acc.at[pl.ds(s * rpw + q * zrows, zrows)])
        plsc.subcore_barrier()

        @pl.loop(0, rh)
        def _(j):
            pltpu.async_copy(ones_v, acc.at[dst_v.at[j]], sem, add=True).wait()

        plsc.subcore_barrier()
        pltpu.sync_copy(acc.at[pl.ds(s * rpw, rpw)],
                        out_hbm.at[c, pl.ds(s * rpw, rpw)])

    return pl.kernel(
        body,
        mesh=mesh,
        compiler_params=pltpu.CompilerParams(use_tc_tiling_on_sc=False),
        out_type=jax.ShapeDtypeStruct((NC, n_pad, HCH), F32),
        scratch_types=[
            pltpu.VMEM((rh, WIN), jnp.int32),
            pltpu.VMEM((WIN, HCH), F32),
            pltpu.VMEM((zrows, HCH), F32),
            pltpu.VMEM_SHARED((n_pad, HCH), F32),
            pltpu.SemaphoreType.DMA,
        ],
    )


@functools.lru_cache(maxsize=None)
def _sc_scatter(n_chunks, n_pad, r2):
    mesh = plsc.VectorSubcoreMesh(core_axis_name="c", subcore_axis_name="s")
    rpw = n_pad // NS
    zrows = rpw // 16
    cpc = n_chunks // NC
    n_pieces = r2 // PIECE
    assert r2 % PIECE == 0 and PIECE % NBUF == 0

    def body(src8_hbm, dst_hbm, t8_hbm, out_hbm, *rest):
        src_v, dst_v = rest[0], rest[1]
        bufs = rest[2:2 + NBUF]
        zbuf = rest[2 + NBUF]
        acc = rest[3 + NBUF]
        sems_g = rest[4 + NBUF:4 + 2 * NBUF]
        sems_s = rest[4 + 2 * NBUF:4 + 3 * NBUF]
        c = lax.axis_index("c")
        s = lax.axis_index("s")

        @pl.loop(0, zrows)
        def _(i):
            zbuf[i, pl.ds(0, CH)] = jnp.zeros((CH,), F32)

        for ci_l in range(cpc):
            ci = c * cpc + ci_l
            for q in range(16):
                pltpu.sync_copy(zbuf, acc.at[pl.ds(s * rpw + q * zrows, zrows)])
            plsc.subcore_barrier()

            for piece in range(n_pieces):
                pltpu.sync_copy(src8_hbm.at[ci, s, pl.ds(piece * PIECE, PIECE)],
                                src_v)
                pltpu.sync_copy(dst_hbm.at[s, pl.ds(piece * PIECE, PIECE)],
                                dst_v)

                @pl.loop(0, PIECE, step=NBUF)
                def _(j):
                    @pl.when(j > 0)
                    def _():
                        for b in range(NBUF):
                            pltpu.make_async_copy(
                                bufs[b], acc.at[dst_v.at[j + b]],
                                sems_s[b]).wait()
                    gathers = [
                        pltpu.async_copy(t8_hbm.at[src_v.at[j + b]], bufs[b],
                                         sems_g[b])
                        for b in range(NBUF)
                    ]
                    for b in range(NBUF):
                        gathers[b].wait()
                        pltpu.async_copy(bufs[b], acc.at[dst_v.at[j + b]],
                                         sems_s[b], add=True)

                for b in range(NBUF):
                    pltpu.make_async_copy(
                        bufs[b], acc.at[dst_v.at[PIECE - NBUF + b]],
                        sems_s[b]).wait()

            plsc.subcore_barrier()
            pltpu.sync_copy(acc.at[pl.ds(s * rpw, rpw)],
                            out_hbm.at[pl.ds(s * rpw, rpw),
                                       pl.ds(ci * CH, CH)])
            if ci_l + 1 < cpc:
                plsc.subcore_barrier()

    return pl.kernel(
        body,
        mesh=mesh,
        compiler_params=pltpu.CompilerParams(use_tc_tiling_on_sc=False),
        out_type=jax.ShapeDtypeStruct((n_pad, n_chunks * CH), F32),
        scratch_types=[
            pltpu.VMEM((PIECE, WIN), jnp.int32),
            pltpu.VMEM((PIECE, WIN), jnp.int32),
        ] + [pltpu.VMEM((WIN, CH), F32)] * NBUF + [
            pltpu.VMEM((zrows, CH), F32),
            pltpu.VMEM_SHARED((n_pad, CH), F32),
        ] + [pltpu.SemaphoreType.DMA] * (2 * NBUF),
    )


def _dot(a, b):
    return lax.dot_general(a, b, (((1,), (0,)), ((), ())),
                           precision=lax.Precision.HIGHEST,
                           preferred_element_type=F32)


def _tc_prep(hist, x_p, n_real):
    n_pad, cin = x_p.shape
    nb = n_pad // BLK

    def body(hist_ref, x_ref, dis_ref, t_ref):
        i = pl.program_id(0)
        deg = hist_ref[0] + hist_ref[1] + 1.0
        row = i * BLK + lax.broadcasted_iota(jnp.int32, (BLK, HCH), 0)
        dis = jnp.where(row < n_real, lax.rsqrt(deg), 0.0)
        dis_col = dis[:, 0:1]
        dis_ref[...] = dis_col
        t_ref[...] = x_ref[...] * dis_col

    return pl.pallas_call(
        body,
        grid=(nb,),
        in_specs=[
            pl.BlockSpec((NC, BLK, HCH), lambda i: (0, i, 0)),
            pl.BlockSpec((BLK, cin), lambda i: (i, 0)),
        ],
        out_specs=[pl.BlockSpec((BLK, 1), lambda i: (i, 0)),
                   pl.BlockSpec((BLK, cin), lambda i: (i, 0))],
        out_shape=[jax.ShapeDtypeStruct((n_pad, 1), F32),
                   jax.ShapeDtypeStruct((n_pad, cin), F32)],
    )(hist, x_p)


def _tc_layer(s_full, t_prev, dis, b, W_next):
    n_pad, fin = t_prev.shape
    fout = W_next.shape[1]
    nb = n_pad // BLK

    def body(s_ref, t_ref, dis_ref, b_ref, w_ref, out_ref):
        dis = dis_ref[...]
        h = jnp.maximum(dis * (s_ref[...] + t_ref[...]) + b_ref[...], 0.0)
        out_ref[...] = dis * _dot(h, w_ref[...])

    return pl.pallas_call(
        body,
        grid=(nb,),
        in_specs=[
            pl.BlockSpec((BLK, fin), lambda i: (i, 0)),
            pl.BlockSpec((BLK, fin), lambda i: (i, 0)),
            pl.BlockSpec((BLK, 1), lambda i: (i, 0)),
            pl.BlockSpec((1, fin), lambda i: (0, 0)),
            pl.BlockSpec((fin, fout), lambda i: (0, 0)),
        ],
        out_specs=pl.BlockSpec((BLK, fout), lambda i: (i, 0)),
        out_shape=jax.ShapeDtypeStruct((n_pad, fout), F32),
    )(s_full, t_prev, dis, b.reshape(1, -1), W_next)


def _tc_layer1(s_full, t0, dis, W1, b1, W2):
    n_pad, cin = t0.shape
    hid = W1.shape[1]
    mid = W2.shape[1]
    nb = n_pad // BLK

    def body(s_ref, t_ref, dis_ref, w1_ref, b1_ref, w2_ref, out_ref):
        dis = dis_ref[...]
        p0 = dis * (s_ref[...] + t_ref[...])
        h1 = jnp.maximum(_dot(p0, w1_ref[...]) + b1_ref[...], 0.0)
        out_ref[...] = dis * _dot(h1, w2_ref[...])

    return pl.pallas_call(
        body,
        grid=(nb,),
        in_specs=[
            pl.BlockSpec((BLK, cin), lambda i: (i, 0)),
            pl.BlockSpec((BLK, cin), lambda i: (i, 0)),
            pl.BlockSpec((BLK, 1), lambda i: (i, 0)),
            pl.BlockSpec((cin, hid), lambda i: (0, 0)),
            pl.BlockSpec((1, hid), lambda i: (0, 0)),
            pl.BlockSpec((hid, mid), lambda i: (0, 0)),
        ],
        out_specs=pl.BlockSpec((BLK, mid), lambda i: (i, 0)),
        out_shape=jax.ShapeDtypeStruct((n_pad, mid), F32),
    )(s_full, t0, dis, W1, b1.reshape(1, -1), W2)


def _tc_final(s_full, t2, dis, b3):
    n_pad, fout = t2.shape
    nb = n_pad // BLK

    def body(s_ref, t_ref, dis_ref, b_ref, out_ref):
        dis = dis_ref[...]
        out_ref[...] = jnp.maximum(dis * (s_ref[...] + t_ref[...]) + b_ref[...],
                                   0.0)

    return pl.pallas_call(
        body,
        grid=(nb,),
        in_specs=[
            pl.BlockSpec((BLK, fout), lambda i: (i, 0)),
            pl.BlockSpec((BLK, fout), lambda i: (i, 0)),
            pl.BlockSpec((BLK, 1), lambda i: (i, 0)),
            pl.BlockSpec((1, fout), lambda i: (0, 0)),
        ],
        out_specs=pl.BlockSpec((BLK, fout), lambda i: (i, 0)),
        out_shape=jax.ShapeDtypeStruct((n_pad, fout), F32),
    )(s_full, t2, dis, b3.reshape(1, -1))


def kernel(x, edge_index, W1, b1, W2, b2, W3, b3):
    n, cin = x.shape
    e = edge_index.shape[1]

    r2 = _cdiv(_cdiv(e, NS * WIN), PIECE) * PIECE
    e_pad = NS * r2 * WIN
    n_pad = (_cdiv(n + 1, BLK)) * BLK

    src = edge_index[0].astype(jnp.int32)
    dst = edge_index[1].astype(jnp.int32)
    pad = jnp.full((e_pad - e,), n, jnp.int32)
    src_p = jnp.concatenate([src, pad])
    dst2 = jnp.concatenate([dst, pad]).reshape(NS, r2, WIN)
    nch = cin // CH
    src8 = (src_p * nch)[None, :] + jnp.arange(nch, dtype=jnp.int32)[:, None]
    src8 = src8.reshape(nch, NS, r2, WIN)
    nch3 = (W3.shape[1]) // CH
    src4 = (src_p * nch3)[None, :] + jnp.arange(nch3, dtype=jnp.int32)[:, None]
    src4 = src4.reshape(nch3, NS, r2, WIN)
    x_p = jnp.pad(x, ((0, n_pad - n), (0, 0)))

    def scat(t, n_chunks, srcb_slabs):
        t8 = t.reshape(n_pad * (t.shape[1] // CH), CH)
        return _sc_scatter(n_chunks, n_pad, r2)(srcb_slabs, dst2, t8)

    hist = _sc_hist(n_pad, r2)(dst2)
    dis, t0 = _tc_prep(hist, x_p, n)
    s0 = scat(t0, nch, src8)
    t1 = _tc_layer1(s0, t0, dis, W1, b1, W2)
    s1 = scat(t1, nch, src8)
    t2 = _tc_layer(s1, t1, dis, b2, W3)
    s2 = scat(t2, nch3, src4)
    out = _tc_final(s2, t2, dis, b3)
    return out[:n]


# speedup vs baseline: 17.1887x; 1.1048x over previous
import functools

import jax
import jax.numpy as jnp
from jax import lax
from jax.experimental import pallas as pl
from jax.experimental.pallas import tpu as pltpu
from jax.experimental.pallas import tpu_sc as plsc

F32 = jnp.float32

NC = 2
NS = 16
WIN = 128
BLK = 512
CH = 32
HCH = 16
NBUF = 4
PIECE = 28


def _cdiv(a, b):
    return (a + b - 1) // b


@functools.lru_cache(maxsize=None)
def _sc_hist(n_pad, r2):
    mesh = plsc.VectorSubcoreMesh(core_axis_name="c", subcore_axis_name="s")
    rpw = n_pad // NS
    zrows = rpw // 4
    rh = r2 // 2

    def body(dst_hbm, out_hbm, dst_v, ones_v, zbuf, acc, sem):
        c = lax.axis_index("c")
        s = lax.axis_index("s")
        pltpu.sync_copy(dst_hbm.at[s, pl.ds(c * rh, rh)], dst_v)

        @pl.loop(0, WIN)
        def _(i):
            ones_v[i, pl.ds(0, HCH)] = jnp.ones((HCH,), F32)

        @pl.loop(0, zrows)
        def _(i):
            zbuf[i, pl.ds(0, HCH)] = jnp.zeros((HCH,), F32)

        for q in range(4):
            pltpu.sync_copy(zbuf, acc.at[pl.ds(s * rpw + q * zrows, zrows)])
        plsc.subcore_barrier()

        @pl.loop(0, rh)
        def _(j):
            pltpu.async_copy(ones_v, acc.at[dst_v.at[j]], sem, add=True).wait()

        plsc.subcore_barrier()
        pltpu.sync_copy(acc.at[pl.ds(s * rpw, rpw)],
                        out_hbm.at[c, pl.ds(s * rpw, rpw)])

    return pl.kernel(
        body,
        mesh=mesh,
        compiler_params=pltpu.CompilerParams(use_tc_tiling_on_sc=False),
        out_type=jax.ShapeDtypeStruct((NC, n_pad, HCH), F32),
        scratch_types=[
            pltpu.VMEM((rh, WIN), jnp.int32),
            pltpu.VMEM((WIN, HCH), F32),
            pltpu.VMEM((zrows, HCH), F32),
            pltpu.VMEM_SHARED((n_pad, HCH), F32),
            pltpu.SemaphoreType.DMA,
        ],
    )


@functools.lru_cache(maxsize=None)
def _sc_scatter(n_chunks, n_pad, r2):
    mesh = plsc.VectorSubcoreMesh(core_axis_name="c", subcore_axis_name="s")
    rpw = n_pad // NS
    zrows = rpw // 16
    cpc = n_chunks // NC
    n_pieces = r2 // PIECE
    assert r2 % PIECE == 0 and PIECE % NBUF == 0

    def body(src8_hbm, dst_hbm, t8_hbm, out_hbm, *rest):
        src_v, dst_v = rest[0], rest[1]
        bufs = rest[2:2 + NBUF]
        zbuf = rest[2 + NBUF]
        acc = rest[3 + NBUF]
        sems_g = rest[4 + NBUF:4 + 2 * NBUF]
        sems_s = rest[4 + 2 * NBUF:4 + 3 * NBUF]
        c = lax.axis_index("c")
        s = lax.axis_index("s")

        @pl.loop(0, zrows)
        def _(i):
            zbuf[i, pl.ds(0, CH)] = jnp.zeros((CH,), F32)

        for ci_l in range(cpc):
            ci = c * cpc + ci_l
            for q in range(16):
                pltpu.sync_copy(zbuf, acc.at[pl.ds(s * rpw + q * zrows, zrows)])
            plsc.subcore_barrier()

            for piece in range(n_pieces):
                pltpu.sync_copy(src8_hbm.at[ci, s, pl.ds(piece * PIECE, PIECE)],
                                src_v)
                pltpu.sync_copy(dst_hbm.at[s, pl.ds(piece * PIECE, PIECE)],
                                dst_v)

                @pl.loop(0, PIECE, step=NBUF)
                def _(j):
                    @pl.when(j > 0)
                    def _():
                        for b in range(NBUF):
                            pltpu.make_async_copy(
                                bufs[b], acc.at[dst_v.at[j + b]],
                                sems_s[b]).wait()
                    gathers = [
                        pltpu.async_copy(t8_hbm.at[src_v.at[j + b]], bufs[b],
                                         sems_g[b])
                        for b in range(NBUF)
                    ]
                    for b in range(NBUF):
                        gathers[b].wait()
                        pltpu.async_copy(bufs[b], acc.at[dst_v.at[j + b]],
                                         sems_s[b], add=True)

                for b in range(NBUF):
                    pltpu.make_async_copy(
                        bufs[b], acc.at[dst_v.at[PIECE - NBUF + b]],
                        sems_s[b]).wait()

            plsc.subcore_barrier()
            pltpu.sync_copy(acc.at[pl.ds(s * rpw, rpw)],
                            out_hbm.at[pl.ds(s * rpw, rpw),
                                       pl.ds(ci * CH, CH)])
            if ci_l + 1 < cpc:
                plsc.subcore_barrier()

    return pl.kernel(
        body,
        mesh=mesh,
        compiler_params=pltpu.CompilerParams(use_tc_tiling_on_sc=False),
        out_type=jax.ShapeDtypeStruct((n_pad, n_chunks * CH), F32),
        scratch_types=[
            pltpu.VMEM((PIECE, WIN), jnp.int32),
            pltpu.VMEM((PIECE, WIN), jnp.int32),
        ] + [pltpu.VMEM((WIN, CH), F32)] * NBUF + [
            pltpu.VMEM((zrows, CH), F32),
            pltpu.VMEM_SHARED((n_pad, CH), F32),
        ] + [pltpu.SemaphoreType.DMA] * (2 * NBUF),
    )


def _dot(a, b):
    return lax.dot_general(a, b, (((1,), (0,)), ((), ())),
                           precision=lax.Precision.HIGHEST,
                           preferred_element_type=F32)


def _tc_prep(hist, x_p, n_real):
    n_pad, cin = x_p.shape
    nb = n_pad // BLK

    def body(hist_ref, x_ref, dis_ref, t_ref):
        i = pl.program_id(0)
        deg = hist_ref[0] + hist_ref[1] + 1.0
        row = i * BLK + lax.broadcasted_iota(jnp.int32, (BLK, HCH), 0)
        dis = jnp.where(row < n_real, lax.rsqrt(deg), 0.0)
        dis_col = dis[:, 0:1]
        dis_ref[...] = dis_col
        t_ref[...] = x_ref[...] * dis_col

    return pl.pallas_call(
        body,
        grid=(nb,),
        in_specs=[
            pl.BlockSpec((NC, BLK, HCH), lambda i: (0, i, 0)),
            pl.BlockSpec((BLK, cin), lambda i: (i, 0)),
        ],
        out_specs=[pl.BlockSpec((BLK, 1), lambda i: (i, 0)),
                   pl.BlockSpec((BLK, cin), lambda i: (i, 0))],
        out_shape=[jax.ShapeDtypeStruct((n_pad, 1), F32),
                   jax.ShapeDtypeStruct((n_pad, cin), F32)],
    )(hist, x_p)


def _tc_layer(s_full, t_prev, dis, b, W_next):
    n_pad, fin = t_prev.shape
    fout = W_next.shape[1]
    nb = n_pad // BLK

    def body(s_ref, t_ref, dis_ref, b_ref, w_ref, out_ref):
        dis = dis_ref[...]
        h = jnp.maximum(dis * (s_ref[...] + t_ref[...]) + b_ref[...], 0.0)
        out_ref[...] = dis * _dot(h, w_ref[...])

    return pl.pallas_call(
        body,
        grid=(nb,),
        in_specs=[
            pl.BlockSpec((BLK, fin), lambda i: (i, 0)),
            pl.BlockSpec((BLK, fin), lambda i: (i, 0)),
            pl.BlockSpec((BLK, 1), lambda i: (i, 0)),
            pl.BlockSpec((1, fin), lambda i: (0, 0)),
            pl.BlockSpec((fin, fout), lambda i: (0, 0)),
        ],
        out_specs=pl.BlockSpec((BLK, fout), lambda i: (i, 0)),
        out_shape=jax.ShapeDtypeStruct((n_pad, fout), F32),
    )(s_full, t_prev, dis, b.reshape(1, -1), W_next)


def _tc_layer1(s_full, t0, dis, W1, b1, W2):
    n_pad, cin = t0.shape
    hid = W1.shape[1]
    mid = W2.shape[1]
    nb = n_pad // BLK

    def body(s_ref, t_ref, dis_ref, w1_ref, b1_ref, w2_ref, out_ref):
        dis = dis_ref[...]
        p0 = dis * (s_ref[...] + t_ref[...])
        h1 = jnp.maximum(_dot(p0, w1_ref[...]) + b1_ref[...], 0.0)
        out_ref[...] = dis * _dot(h1, w2_ref[...])

    return pl.pallas_call(
        body,
        grid=(nb,),
        in_specs=[
            pl.BlockSpec((BLK, cin), lambda i: (i, 0)),
            pl.BlockSpec((BLK, cin), lambda i: (i, 0)),
            pl.BlockSpec((BLK, 1), lambda i: (i, 0)),
            pl.BlockSpec((cin, hid), lambda i: (0, 0)),
            pl.BlockSpec((1, hid), lambda i: (0, 0)),
            pl.BlockSpec((hid, mid), lambda i: (0, 0)),
        ],
        out_specs=pl.BlockSpec((BLK, mid), lambda i: (i, 0)),
        out_shape=jax.ShapeDtypeStruct((n_pad, mid), F32),
    )(s_full, t0, dis, W1, b1.reshape(1, -1), W2)


def _tc_final(s_full, t2, dis, b3):
    n_pad, fout = t2.shape
    nb = n_pad // BLK

    def body(s_ref, t_ref, dis_ref, b_ref, out_ref):
        dis = dis_ref[...]
        out_ref[...] = jnp.maximum(dis * (s_ref[...] + t_ref[...]) + b_ref[...],
                                   0.0)

    return pl.pallas_call(
        body,
        grid=(nb,),
        in_specs=[
            pl.BlockSpec((BLK, fout), lambda i: (i, 0)),
            pl.BlockSpec((BLK, fout), lambda i: (i, 0)),
            pl.BlockSpec((BLK, 1), lambda i: (i, 0)),
            pl.BlockSpec((1, fout), lambda i: (0, 0)),
        ],
        out_specs=pl.BlockSpec((BLK, fout), lambda i: (i, 0)),
        out_shape=jax.ShapeDtypeStruct((n_pad, fout), F32),
    )(s_full, t2, dis, b3.reshape(1, -1))


def kernel(x, edge_index, W1, b1, W2, b2, W3, b3):
    n, cin = x.shape
    e = edge_index.shape[1]

    r2 = _cdiv(_cdiv(e, NS * WIN), PIECE) * PIECE
    e_pad = NS * r2 * WIN
    n_pad = (_cdiv(n + 1, BLK)) * BLK

    src = edge_index[0].astype(jnp.int32)
    dst = edge_index[1].astype(jnp.int32)
    pad = jnp.full((e_pad - e,), n, jnp.int32)
    src_p = jnp.concatenate([src, pad])
    dst2 = jnp.concatenate([dst, pad]).reshape(NS, r2, WIN)
    nch = cin // CH
    src8 = (src_p * nch)[None, :] + jnp.arange(nch, dtype=jnp.int32)[:, None]
    src8 = src8.reshape(nch, NS, r2, WIN)
    nch3 = (W3.shape[1]) // CH
    src4 = (src_p * nch3)[None, :] + jnp.arange(nch3, dtype=jnp.int32)[:, None]
    src4 = src4.reshape(nch3, NS, r2, WIN)
    x_p = jnp.pad(x, ((0, n_pad - n), (0, 0)))

    def scat(t, n_chunks, srcb_slabs):
        t8 = t.reshape(n_pad * (t.shape[1] // CH), CH)
        return _sc_scatter(n_chunks, n_pad, r2)(srcb_slabs, dst2, t8)

    hist = _sc_hist(n_pad, r2)(dst2)
    dis, t0 = _tc_prep(hist, x_p, n)
    s0 = scat(t0, nch, src8)
    t1 = _tc_layer1(s0, t0, dis, W1, b1, W2)
    s1 = scat(t1, nch, src8)
    t2 = _tc_layer(s1, t1, dis, b2, W3)
    s2 = scat(t2, nch3, src4)
    out = _tc_final(s2, t2, dis, b3)
    return out[:n]
